# baseline (speedup 1.0000x reference)
import functools

import jax
import jax.numpy as jnp
from jax.experimental import pallas as pl
from jax.experimental.pallas import tpu as pltpu

B, N1, N2, F = 10000, 8, 8, 128
AGG, OUT, LBL = 128, 128, 50
BB = 1000


def _l2norm(x):
    s = jnp.sum(x * x, axis=-1, keepdims=True)
    return x * jax.lax.rsqrt(jnp.maximum(s, 1e-12))


def _fused_kernel(hop2_ref, hop1_ref, target_ref,
                  wagg0_ref, wagg1_ref,
                  wc0x_ref, wc0a_ref, wc1t_ref, wc1a_ref,
                  wcls_ref, out_ref, acc_ref):
    dot = functools.partial(jnp.dot, preferred_element_type=jnp.float32)
    j = pl.program_id(1)

    x2 = hop2_ref[...].reshape(BB * N1, F)
    part = jax.nn.relu(dot(x2, wagg0_ref[...]))

    @pl.when(j == 0)
    def _init():
        acc_ref[...] = part

    @pl.when(j != 0)
    def _acc():
        acc_ref[...] += part

    @pl.when(j == N2 - 1)
    def _tail():
        a_h2 = acc_ref[...]
        wagg0 = wagg0_ref[...]

        hop1 = hop1_ref[...].reshape(BB * N1, F)
        h1 = _l2norm(jax.nn.relu(dot(hop1, wc0x_ref[...]) + dot(a_h2, wc0a_ref[...])))

        a_h1 = jnp.sum(jax.nn.relu(dot(hop1, wagg0)).reshape(BB, N1, AGG), axis=1)

        t = _l2norm(jax.nn.relu(dot(target_ref[...], wc0x_ref[...]) + dot(a_h1, wc0a_ref[...])))

        a_l1 = jnp.sum(jax.nn.relu(dot(h1, wagg1_ref[...])).reshape(BB, N1, AGG), axis=1)

        full = _l2norm(dot(t, wc1t_ref[...]) + dot(a_l1, wc1a_ref[...]))
        full = _l2norm(full)

        out_ref[...] = jax.nn.relu(dot(full, wcls_ref[...]))


def kernel(hop2, hop1, target, W_agg0, W_agg1, W_comb0, W_comb1, W_cls):
    wagg0 = W_agg0 * (1.0 / N2)
    wagg1 = W_agg1 * (1.0 / N1)
    wc0x, wc0a = W_comb0[:F], W_comb0[F:]
    wc1t, wc1a = W_comb1[:OUT], W_comb1[OUT:]

    grid = (B // BB, N2)
    full_w = lambda shape: pl.BlockSpec(shape, lambda i, j: (0,) * len(shape))
    out = pl.pallas_call(
        _fused_kernel,
        grid=grid,
        in_specs=[
            pl.BlockSpec((BB, N1, 1, 1, F), lambda i, j: (i, 0, j, 0, 0)),
            pl.BlockSpec((BB, N1, F), lambda i, j: (i, 0, 0)),
            pl.BlockSpec((BB, F), lambda i, j: (i, 0)),
            full_w((F, AGG)),
            full_w((OUT, AGG)),
            full_w((F, OUT)),
            full_w((AGG, OUT)),
            full_w((OUT, OUT)),
            full_w((AGG, OUT)),
            full_w((OUT, LBL)),
        ],
        out_specs=pl.BlockSpec((BB, LBL), lambda i, j: (i, 0)),
        out_shape=jax.ShapeDtypeStruct((B, LBL), jnp.float32),
        scratch_shapes=[pltpu.VMEM((BB * N1, AGG), jnp.float32)],
        compiler_params=pltpu.CompilerParams(
            dimension_semantics=("arbitrary", "arbitrary"),
        ),
    )(hop2.reshape(B, N1, N2, 1, F), hop1, target,
      wagg0, wagg1, wc0x, wc0a, wc1t, wc1a, W_cls)
    return out
